# baseline (speedup 1.0000x reference)
import jax
import jax.numpy as jnp
from jax import lax
from jax.experimental import pallas as pl
from jax.experimental.pallas import tpu as pltpu
from jax.experimental.pallas import tpu_sc as plsc

N = 10000
E = 320000
H = 128
F = 128
G = 50

NC = 2
NS = 16
L = 16
NW = NC * NS
EPW = E // NW
K = 80
NCHUNK = EPW // K
ROW_SPAN = 624
TAIL = N - NS * ROW_SPAN
TAIL_OFF = NS * ROW_SPAN

EB = 6400
NB = 2000


def _wmlp_body(rbft_ref, wm1_ref, bm1_ref, wm2_ref, bm2_ref, out_ref):
    h = lax.dot_general(
        rbft_ref[...], wm1_ref[...], (((0,), (0,)), ((), ())),
        preferred_element_type=jnp.float32,
    )
    h = h + bm1_ref[...]
    h = h * jax.nn.sigmoid(h)
    out_ref[...] = (
        jnp.dot(h, wm2_ref[...], preferred_element_type=jnp.float32) + bm2_ref[...]
    )


def _lin1_body(x_ref, wl1_ref, bl1_ref, out_ref):
    out_ref[...] = (
        jnp.dot(x_ref[...], wl1_ref[...], preferred_element_type=jnp.float32)
        + bl1_ref[...]
    )


def _final_body(agg_ref, wl2_ref, bl2_ref, out_ref):
    a = agg_ref[0] + agg_ref[1]
    t = jnp.dot(a, wl2_ref[...], preferred_element_type=jnp.float32) + bl2_ref[...]
    out_ref[...] = t * jax.nn.sigmoid(t)


def _when(cond, f):
    if isinstance(cond, bool):
        if cond:
            f()
    else:
        pl.when(cond)(f)


def _sc_body(y_hbm, rc_hbm, cut_hbm, w_hbm, out_hbm,
             rc0, rc1, rc2, cv0, cv1, cv2, ybuf0, ybuf1, wbuf0, wbuf1, aggs,
             sem0, sem1, semi0, semi1, semi2, semsc0, semsc1):
    c = lax.axis_index("c")
    s = lax.axis_index("s")
    w = c * NS + s
    rcs = (rc0, rc1, rc2)
    cvs = (cv0, cv1, cv2)
    ybufs = (ybuf0, ybuf1)
    wbufs = (wbuf0, wbuf1)
    sems = (sem0, sem1)
    semis = (semi0, semi1, semi2)
    semscs = (semsc0, semsc1)

    zero16 = jnp.zeros((L,), jnp.float32)

    def zrow(r, _):
        for cc in range(F // L):
            ybuf0[r, pl.ds(cc * L, L)] = zero16
        return 0

    lax.fori_loop(0, K, zrow, 0)
    for i in range(ROW_SPAN // K):
        pltpu.sync_copy(ybuf0, aggs.at[pl.ds(s * ROW_SPAN + i * K, K)])
    _rem = ROW_SPAN - (ROW_SPAN // K) * K
    pltpu.sync_copy(
        ybuf0.at[pl.ds(0, _rem)],
        aggs.at[pl.ds(s * ROW_SPAN + (ROW_SPAN // K) * K, _rem)],
    )

    @pl.when(s == NS - 1)
    def _zero_tail():
        pltpu.sync_copy(ybuf0.at[pl.ds(0, TAIL)], aggs.at[pl.ds(TAIL_OFF, TAIL)])

    plsc.subcore_barrier()

    cbase = w * NCHUNK

    def idx_issue(j, ib):
        pltpu.async_copy(rc_hbm.at[cbase + j], rcs[ib], semis[ib])
        pltpu.async_copy(cut_hbm.at[cbase + j], cvs[ib], semis[ib])

    def idx_wait(j, ib):
        pltpu.make_async_copy(rc_hbm.at[cbase + j], rcs[ib], semis[ib]).wait()
        pltpu.make_async_copy(cut_hbm.at[cbase + j], cvs[ib], semis[ib]).wait()

    def gw_issue(j, db, ib):
        pltpu.async_copy(y_hbm.at[rcs[ib].at[0]], ybufs[db], sems[db])
        pltpu.async_copy(
            w_hbm.at[pl.ds((cbase + j) * K, K)], wbufs[db], sems[db]
        )

    def gw_wait(j, db, ib):
        pltpu.make_async_copy(y_hbm.at[rcs[ib].at[0]], ybufs[db], sems[db]).wait()
        pltpu.make_async_copy(
            w_hbm.at[pl.ds((cbase + j) * K, K)], wbufs[db], sems[db]
        ).wait()

    def sc_issue(j, db, ib):
        pltpu.async_copy(ybufs[db], aggs.at[rcs[ib].at[1]], semscs[db], add=True)

    def sc_wait(j, db, ib):
        pltpu.make_async_copy(ybufs[db], aggs.at[rcs[ib].at[1]], semscs[db]).wait()

    def do_chunk(j, db, ib):
        gw_wait(j, db, ib)
        _when(j + 1 < NCHUNK, lambda: idx_wait(j + 1, (ib + 1) % 3))
        _when(j >= 1, lambda: sc_wait(j - 1, 1 - db, (ib + 2) % 3))
        _when(j + 1 < NCHUNK, lambda: gw_issue(j + 1, 1 - db, (ib + 1) % 3))
        _when(j + 2 < NCHUNK, lambda: idx_issue(j + 2, (ib + 2) % 3))

        yb, wb, cvb = ybufs[db], wbufs[db], cvs[ib]

        def mulrow(r, _):
            g16 = (r // L) * L
            lane = r - g16
            cut16 = cvb[0, pl.ds(g16, L)]
            cv = lax.gather(
                cut16,
                jnp.full((L, 1), lane, jnp.int32),
                dimension_numbers=lax.GatherDimensionNumbers(
                    offset_dims=(), collapsed_slice_dims=(0,),
                    start_index_map=(0,)),
                slice_sizes=(1,),
                mode=lax.GatherScatterMode.PROMISE_IN_BOUNDS,
            )
            for cc in range(F // L):
                sl = pl.ds(cc * L, L)
                yb[r, sl] = yb[r, sl] * wb[r, sl] * cv
            return 0

        lax.fori_loop(0, K, mulrow, 0)
        sc_issue(j, db, ib)

    idx_issue(0, 0)
    idx_wait(0, 0)
    gw_issue(0, 0, 0)
    idx_issue(1, 1)

    def six(p, _):
        base = 6 * p
        for i in range(6):
            do_chunk(base + i, i % 2, i % 3)
        return 0

    lax.fori_loop(0, NCHUNK // 6, six, 0)
    for j in range((NCHUNK // 6) * 6, NCHUNK):
        do_chunk(j, j % 2, j % 3)
    sc_wait(NCHUNK - 1, (NCHUNK - 1) % 2, (NCHUNK - 1) % 3)

    plsc.subcore_barrier()

    pltpu.sync_copy(
        aggs.at[pl.ds(s * ROW_SPAN, ROW_SPAN)],
        out_hbm.at[c, pl.ds(s * ROW_SPAN, ROW_SPAN)],
    )

    @pl.when(s == NS - 1)
    def _write_tail():
        pltpu.sync_copy(
            aggs.at[pl.ds(TAIL_OFF, TAIL)],
            out_hbm.at[c, pl.ds(TAIL_OFF, TAIL)],
        )


def kernel(x, edge_index, rbf, cutoff_val, Wm1, bm1, Wm2, bm2, Wl1, bl1, Wl2, bl2):
    row = edge_index[0]
    col = edge_index[1]
    rbft = rbf.T
    rc3 = jnp.stack([col.reshape(E // K, K), row.reshape(E // K, K)], axis=1)
    cut3 = cutoff_val.reshape(E // K, 1, K)

    W = pl.pallas_call(
        _wmlp_body,
        grid=(E // EB,),
        in_specs=[
            pl.BlockSpec((G, EB), lambda i: (0, i)),
            pl.BlockSpec((G, F), lambda i: (0, 0)),
            pl.BlockSpec((1, F), lambda i: (0, 0)),
            pl.BlockSpec((F, F), lambda i: (0, 0)),
            pl.BlockSpec((1, F), lambda i: (0, 0)),
        ],
        out_specs=pl.BlockSpec((EB, F), lambda i: (i, 0)),
        out_shape=jax.ShapeDtypeStruct((E, F), jnp.float32),
    )(rbft, Wm1, bm1.reshape(1, F), Wm2, bm2.reshape(1, F))

    y = pl.pallas_call(
        _lin1_body,
        grid=(N // NB,),
        in_specs=[
            pl.BlockSpec((NB, H), lambda i: (i, 0)),
            pl.BlockSpec((H, F), lambda i: (0, 0)),
            pl.BlockSpec((1, F), lambda i: (0, 0)),
        ],
        out_specs=pl.BlockSpec((NB, F), lambda i: (i, 0)),
        out_shape=jax.ShapeDtypeStruct((N, F), jnp.float32),
    )(x, Wl1, bl1.reshape(1, F))

    sc_scatter = pl.kernel(
        _sc_body,
        out_type=jax.ShapeDtypeStruct((NC, N, F), jnp.float32),
        mesh=plsc.VectorSubcoreMesh(core_axis_name="c", subcore_axis_name="s"),
        scratch_types=[
            pltpu.VMEM((2, K), jnp.int32),
            pltpu.VMEM((2, K), jnp.int32),
            pltpu.VMEM((2, K), jnp.int32),
            pltpu.VMEM((1, K), jnp.float32),
            pltpu.VMEM((1, K), jnp.float32),
            pltpu.VMEM((1, K), jnp.float32),
            pltpu.VMEM((K, F), jnp.float32),
            pltpu.VMEM((K, F), jnp.float32),
            pltpu.VMEM((K, F), jnp.float32),
            pltpu.VMEM((K, F), jnp.float32),
            pltpu.VMEM_SHARED((N, F), jnp.float32),
            pltpu.SemaphoreType.DMA,
            pltpu.SemaphoreType.DMA,
            pltpu.SemaphoreType.DMA,
            pltpu.SemaphoreType.DMA,
            pltpu.SemaphoreType.DMA,
            pltpu.SemaphoreType.DMA,
            pltpu.SemaphoreType.DMA,
        ],
    )
    aggp = sc_scatter(y, rc3, cut3, W)

    out = pl.pallas_call(
        _final_body,
        grid=(N // NB,),
        in_specs=[
            pl.BlockSpec((NC, NB, F), lambda i: (0, i, 0)),
            pl.BlockSpec((F, H), lambda i: (0, 0)),
            pl.BlockSpec((1, H), lambda i: (0, 0)),
        ],
        out_specs=pl.BlockSpec((NB, H), lambda i: (i, 0)),
        out_shape=jax.ShapeDtypeStruct((N, H), jnp.float32),
    )(aggp, Wl2, bl2.reshape(1, H))
    return out


# speedup vs baseline: 1.4967x; 1.4967x over previous
import jax
import jax.numpy as jnp
from jax import lax
from jax.experimental import pallas as pl
from jax.experimental.pallas import tpu as pltpu
from jax.experimental.pallas import tpu_sc as plsc

N = 10000
E = 320000
H = 128
F = 128
G = 50

NC = 2
NS = 16
L = 16
NW = NC * NS
EPW = E // NW
K = 80
NCHUNK = EPW // K
ROW_SPAN = 624
TAIL = N - NS * ROW_SPAN
TAIL_OFF = NS * ROW_SPAN

EB = 6400
NB = 2000


def _wmlp_body(rbft_ref, wm1_ref, bm1_ref, wm2_ref, bm2_ref, out_ref):
    h = lax.dot_general(
        rbft_ref[...], wm1_ref[...], (((0,), (0,)), ((), ())),
        preferred_element_type=jnp.float32,
    )
    h = h + bm1_ref[...]
    h = h * jax.nn.sigmoid(h)
    out_ref[...] = (
        jnp.dot(h, wm2_ref[...], preferred_element_type=jnp.float32) + bm2_ref[...]
    )


def _lin1_body(x_ref, wl1_ref, bl1_ref, out_ref):
    out_ref[...] = (
        jnp.dot(x_ref[...], wl1_ref[...], preferred_element_type=jnp.float32)
        + bl1_ref[...]
    )


def _final_body(agg_ref, wl2_ref, bl2_ref, out_ref):
    a = agg_ref[0] + agg_ref[1]
    t = jnp.dot(a, wl2_ref[...], preferred_element_type=jnp.float32) + bl2_ref[...]
    out_ref[...] = t * jax.nn.sigmoid(t)


def _when(cond, f):
    if isinstance(cond, bool):
        if cond:
            f()
    else:
        pl.when(cond)(f)


def _sc_body(y_hbm, rc_hbm, cut_hbm, w_hbm, out_hbm,
             rc0, rc1, rc2, cv0, cv1, cv2, ybuf0, ybuf1, ybuf2, wbuf, aggs,
             sem0, sem1, sem2, semw, semi0, semi1, semi2,
             semsc0, semsc1, semsc2):
    c = lax.axis_index("c")
    s = lax.axis_index("s")
    w = c * NS + s
    rcs = (rc0, rc1, rc2)
    cvs = (cv0, cv1, cv2)
    ybufs = (ybuf0, ybuf1, ybuf2)
    sems = (sem0, sem1, sem2)
    semis = (semi0, semi1, semi2)
    semscs = (semsc0, semsc1, semsc2)

    zero16 = jnp.zeros((L,), jnp.float32)

    def zrow(r, _):
        for cc in range(F // L):
            ybuf0[r, pl.ds(cc * L, L)] = zero16
        return 0

    lax.fori_loop(0, K, zrow, 0)
    for i in range(ROW_SPAN // K):
        pltpu.sync_copy(ybuf0, aggs.at[pl.ds(s * ROW_SPAN + i * K, K)])
    _rem = ROW_SPAN - (ROW_SPAN // K) * K
    pltpu.sync_copy(
        ybuf0.at[pl.ds(0, _rem)],
        aggs.at[pl.ds(s * ROW_SPAN + (ROW_SPAN // K) * K, _rem)],
    )

    @pl.when(s == NS - 1)
    def _zero_tail():
        pltpu.sync_copy(ybuf0.at[pl.ds(0, TAIL)], aggs.at[pl.ds(TAIL_OFF, TAIL)])

    plsc.subcore_barrier()

    cbase = w * NCHUNK

    def idx_issue(j, ib):
        pltpu.async_copy(rc_hbm.at[cbase + j], rcs[ib], semis[ib])
        pltpu.async_copy(cut_hbm.at[cbase + j], cvs[ib], semis[ib])

    def idx_wait(j, ib):
        pltpu.make_async_copy(rc_hbm.at[cbase + j], rcs[ib], semis[ib]).wait()
        pltpu.make_async_copy(cut_hbm.at[cbase + j], cvs[ib], semis[ib]).wait()

    def g_issue(j, db, ib):
        pltpu.async_copy(y_hbm.at[rcs[ib].at[0]], ybufs[db], sems[db])

    def g_wait(j, db, ib):
        pltpu.make_async_copy(y_hbm.at[rcs[ib].at[0]], ybufs[db], sems[db]).wait()

    def w_issue(j):
        pltpu.async_copy(w_hbm.at[pl.ds((cbase + j) * K, K)], wbuf, semw)

    def w_wait(j):
        pltpu.make_async_copy(
            w_hbm.at[pl.ds((cbase + j) * K, K)], wbuf, semw
        ).wait()

    def sc_issue(j, db, ib):
        pltpu.async_copy(ybufs[db], aggs.at[rcs[ib].at[1]], semscs[db], add=True)

    def sc_wait(j, db, ib):
        pltpu.make_async_copy(ybufs[db], aggs.at[rcs[ib].at[1]], semscs[db]).wait()

    def do_chunk(j, ib):
        g_wait(j, ib, ib)
        w_wait(j)
        _when(j + 1 < NCHUNK, lambda: idx_wait(j + 1, (ib + 1) % 3))
        _when(j >= 1, lambda: sc_wait(j - 1, (ib + 2) % 3, (ib + 2) % 3))
        _when(j + 1 < NCHUNK, lambda: g_issue(j + 1, (ib + 1) % 3, (ib + 1) % 3))
        _when(j + 2 < NCHUNK, lambda: idx_issue(j + 2, (ib + 2) % 3))

        yb, wb, cvb = ybufs[ib], wbuf, cvs[ib]

        @plsc.parallel_loop(0, K // L)
        def mulgroup(g):
            cut16 = cvb[0, pl.ds(g * L, L)]
            for i in range(L):
                r = g * L + i
                cv = jnp.full((L,), cut16[i], jnp.float32)
                for cc in range(F // L):
                    sl = pl.ds(cc * L, L)
                    yb[r, sl] = yb[r, sl] * wb[r, sl] * cv
        sc_issue(j, ib, ib)
        _when(j + 1 < NCHUNK, lambda: w_issue(j + 1))

    idx_issue(0, 0)
    idx_wait(0, 0)
    g_issue(0, 0, 0)
    w_issue(0)
    idx_issue(1, 1)

    def three(p, _):
        base = 3 * p
        for i in range(3):
            do_chunk(base + i, i)
        return 0

    lax.fori_loop(0, NCHUNK // 3, three, 0)
    for j in range((NCHUNK // 3) * 3, NCHUNK):
        do_chunk(j, j % 3)
    sc_wait(NCHUNK - 1, (NCHUNK - 1) % 3, (NCHUNK - 1) % 3)

    plsc.subcore_barrier()

    pltpu.sync_copy(
        aggs.at[pl.ds(s * ROW_SPAN, ROW_SPAN)],
        out_hbm.at[c, pl.ds(s * ROW_SPAN, ROW_SPAN)],
    )

    @pl.when(s == NS - 1)
    def _write_tail():
        pltpu.sync_copy(
            aggs.at[pl.ds(TAIL_OFF, TAIL)],
            out_hbm.at[c, pl.ds(TAIL_OFF, TAIL)],
        )


def kernel(x, edge_index, rbf, cutoff_val, Wm1, bm1, Wm2, bm2, Wl1, bl1, Wl2, bl2):
    row = edge_index[0]
    col = edge_index[1]
    rbft = rbf.T
    rc3 = jnp.stack([col.reshape(E // K, K), row.reshape(E // K, K)], axis=1)
    cut3 = cutoff_val.reshape(E // K, 1, K)

    W = pl.pallas_call(
        _wmlp_body,
        grid=(E // EB,),
        in_specs=[
            pl.BlockSpec((G, EB), lambda i: (0, i)),
            pl.BlockSpec((G, F), lambda i: (0, 0)),
            pl.BlockSpec((1, F), lambda i: (0, 0)),
            pl.BlockSpec((F, F), lambda i: (0, 0)),
            pl.BlockSpec((1, F), lambda i: (0, 0)),
        ],
        out_specs=pl.BlockSpec((EB, F), lambda i: (i, 0)),
        out_shape=jax.ShapeDtypeStruct((E, F), jnp.float32),
    )(rbft, Wm1, bm1.reshape(1, F), Wm2, bm2.reshape(1, F))

    y = pl.pallas_call(
        _lin1_body,
        grid=(N // NB,),
        in_specs=[
            pl.BlockSpec((NB, H), lambda i: (i, 0)),
            pl.BlockSpec((H, F), lambda i: (0, 0)),
            pl.BlockSpec((1, F), lambda i: (0, 0)),
        ],
        out_specs=pl.BlockSpec((NB, F), lambda i: (i, 0)),
        out_shape=jax.ShapeDtypeStruct((N, F), jnp.float32),
    )(x, Wl1, bl1.reshape(1, F))

    sc_scatter = pl.kernel(
        _sc_body,
        out_type=jax.ShapeDtypeStruct((NC, N, F), jnp.float32),
        mesh=plsc.VectorSubcoreMesh(core_axis_name="c", subcore_axis_name="s"),
        scratch_types=[
            pltpu.VMEM((2, K), jnp.int32),
            pltpu.VMEM((2, K), jnp.int32),
            pltpu.VMEM((2, K), jnp.int32),
            pltpu.VMEM((1, K), jnp.float32),
            pltpu.VMEM((1, K), jnp.float32),
            pltpu.VMEM((1, K), jnp.float32),
            pltpu.VMEM((K, F), jnp.float32),
            pltpu.VMEM((K, F), jnp.float32),
            pltpu.VMEM((K, F), jnp.float32),
            pltpu.VMEM((K, F), jnp.float32),
            pltpu.VMEM_SHARED((N, F), jnp.float32),
            pltpu.SemaphoreType.DMA,
            pltpu.SemaphoreType.DMA,
            pltpu.SemaphoreType.DMA,
            pltpu.SemaphoreType.DMA,
            pltpu.SemaphoreType.DMA,
            pltpu.SemaphoreType.DMA,
            pltpu.SemaphoreType.DMA,
            pltpu.SemaphoreType.DMA,
            pltpu.SemaphoreType.DMA,
            pltpu.SemaphoreType.DMA,
        ],
    )
    aggp = sc_scatter(y, rc3, cut3, W)

    out = pl.pallas_call(
        _final_body,
        grid=(N // NB,),
        in_specs=[
            pl.BlockSpec((NC, NB, F), lambda i: (0, i, 0)),
            pl.BlockSpec((F, H), lambda i: (0, 0)),
            pl.BlockSpec((1, H), lambda i: (0, 0)),
        ],
        out_specs=pl.BlockSpec((NB, H), lambda i: (i, 0)),
        out_shape=jax.ShapeDtypeStruct((N, H), jnp.float32),
    )(aggp, Wl2, bl2.reshape(1, H))
    return out
